# Initial kernel scaffold; baseline (speedup 1.0000x reference)
import jax
import jax.numpy as jnp
from jax.experimental import pallas as pl


def kernel(x24, x27, x30, x33):
    raise NotImplementedError("write your pallas kernel here")


# speedup vs baseline: 19.6967x; 19.6967x over previous
import jax
import jax.numpy as jnp
from jax import lax
from jax.experimental import pallas as pl
from jax.experimental.pallas import tpu as pltpu

_CONF = 0.3
_IOU = 0.45
_CAND = 1000
_CANDP = 1024
_MAXDET = 300
_MAXWH = 4096.0
_N = 16320
_NP = 16384


def _detect_kernel(pred_ref, score_ref, cls_ref, box_ref):
    p = pred_ref[...]
    col = lax.broadcasted_iota(jnp.int32, p.shape, 1)
    obj = p[:, 4:5]
    joint = obj * p
    joint = jnp.where((col >= 5) & (col < 85), joint, -jnp.inf)
    maxv = jnp.max(joint, axis=1, keepdims=True)
    cid = jnp.min(jnp.where(joint == maxv, col - 5, 10**9), axis=1,
                  keepdims=True)
    row = lax.broadcasted_iota(jnp.int32, maxv.shape, 0)
    score = jnp.where(maxv > _CONF, maxv, 0.0)
    score_ref[...] = jnp.where(row < _N, score, -1.0)
    cls_ref[...] = cid.astype(jnp.float32)
    x = p[:, 0:1]
    y = p[:, 1:2]
    w = p[:, 2:3]
    h = p[:, 3:4]
    box_ref[...] = jnp.concatenate(
        [x - w / 2.0, y - h / 2.0, x + w / 2.0, y + h / 2.0], axis=1)


def _nms_kernel(ts_ref, bc_ref, br_ref, cc_ref, cr_ref, out_ref, sup_ref):
    offc = cc_ref[...] * _MAXWH
    offr = cr_ref[...] * _MAXWH
    x1c = bc_ref[:, 0:1] + offc
    y1c = bc_ref[:, 1:2] + offc
    x2c = bc_ref[:, 2:3] + offc
    y2c = bc_ref[:, 3:4] + offc
    x1r = br_ref[0:1, :] + offr
    y1r = br_ref[1:2, :] + offr
    x2r = br_ref[2:3, :] + offr
    y2r = br_ref[3:4, :] + offr
    areac = (x2c - x1c) * (y2c - y1c)
    arear = (x2r - x1r) * (y2r - y1r)
    xx1 = jnp.maximum(x1c, x1r)
    yy1 = jnp.maximum(y1c, y1r)
    xx2 = jnp.minimum(x2c, x2r)
    yy2 = jnp.minimum(y2c, y2r)
    w = jnp.maximum(xx2 - xx1, 0.0)
    h = jnp.maximum(yy2 - yy1, 0.0)
    inter = w * h
    iou = inter / (areac + arear - inter + 1e-7)
    sup_ref[...] = jnp.where(iou > _IOU, 1.0, 0.0)
    colr = lax.broadcasted_iota(jnp.int32, (1, _CANDP), 1)

    def body(i, keep):
        rowv = sup_ref[pl.ds(i, 1), :]
        keep_i = jnp.sum(jnp.where(colr == i, keep, 0.0))
        gt = jnp.where(colr > i, 1.0, 0.0)
        return keep * (1.0 - rowv * gt * keep_i)

    keep = lax.fori_loop(0, _CAND, body, jnp.ones((1, _CANDP), jnp.float32))
    ts = ts_ref[...]
    out_ref[...] = jnp.where((keep > 0.5) & (ts > 0.0), ts, 0.0)


def _detect(pred_pad, interpret=False):
    return pl.pallas_call(
        _detect_kernel,
        out_shape=[
            jax.ShapeDtypeStruct((_NP, 1), jnp.float32),
            jax.ShapeDtypeStruct((_NP, 1), jnp.float32),
            jax.ShapeDtypeStruct((_NP, 4), jnp.float32),
        ],
        interpret=interpret,
    )(pred_pad)


def _nms(ts_row, bc, br, cc, cr, interpret=False):
    return pl.pallas_call(
        _nms_kernel,
        out_shape=jax.ShapeDtypeStruct((1, _CANDP), jnp.float32),
        scratch_shapes=[pltpu.VMEM((_CANDP, _CANDP), jnp.float32)],
        interpret=interpret,
    )(ts_row, bc, br, cc, cr)


def _run(x24, x27, x30, x33, interpret=False):
    pred = jnp.concatenate([x24[0], x27[0], x30[0], x33[0]], axis=0)
    pred = jnp.pad(pred, ((0, _NP - _N), (0, 128 - 85)))
    score, cls_f, boxes = _detect(pred, interpret=interpret)
    scores = score[:, 0]
    ts, ti = lax.top_k(scores, _CAND)
    tb = boxes[ti]
    tc = cls_f[ti, 0]
    ts_p = jnp.concatenate([ts, jnp.full((_CANDP - _CAND,), -1.0,
                                         jnp.float32)])
    tb_p = jnp.pad(tb, ((0, _CANDP - _CAND), (0, 0)))
    tc_p = jnp.pad(tc, ((0, _CANDP - _CAND),))
    final = _nms(ts_p[None, :], tb_p, tb_p.T, tc_p[:, None], tc_p[None, :],
                 interpret=interpret)
    final = final[0, :_CAND]
    det_sc, det_i = lax.top_k(final, _MAXDET)
    sel = ti[det_i]
    return jnp.concatenate(
        [boxes[sel], det_sc[:, None], cls_f[sel]], axis=1)


def kernel(x24, x27, x30, x33):
    return _run(x24, x27, x30, x33)


# speedup vs baseline: 62.6706x; 3.1818x over previous
import jax
import jax.numpy as jnp
from jax import lax
from jax.experimental import pallas as pl
from jax.experimental.pallas import tpu as pltpu

_CONF = 0.3
_IOU = 0.45
_CAND = 1000
_CANDP = 1024
_MAXDET = 300
_MAXWH = 4096.0
_N = 16320
_NP = 16384
_PARTS = (12288, 3072, 768, 192)


def _detect_kernel(x24_ref, x27_ref, x30_ref, x33_ref, score_ref, feat_ref):
    off = 0
    for ref, n in zip((x24_ref, x27_ref, x30_ref, x33_ref), _PARTS):
        p = ref[0]
        col = lax.broadcasted_iota(jnp.int32, p.shape, 1)
        obj = p[:, 4:5]
        joint = obj * p
        joint = jnp.where((col >= 5) & (col < 85), joint, -jnp.inf)
        maxv = jnp.max(joint, axis=1, keepdims=True)
        cid = jnp.min(jnp.where(joint == maxv, col - 5, 10**9), axis=1,
                      keepdims=True).astype(jnp.float32)
        score = jnp.where(maxv > _CONF, maxv, 0.0)
        score_ref[off:off + n, :] = score
        x = p[:, 0:1]
        y = p[:, 1:2]
        w = p[:, 2:3]
        h = p[:, 3:4]
        zero = jnp.zeros_like(score)
        feat_ref[off:off + n, :] = jnp.concatenate(
            [x - w / 2.0, y - h / 2.0, x + w / 2.0, y + h / 2.0,
             cid, score, zero, zero], axis=1)
        off += n
    score_ref[_N:, :] = jnp.full((_NP - _N, 1), -1.0, jnp.float32)
    feat_ref[_N:, :] = jnp.zeros((_NP - _N, 8), jnp.float32)


def _nms_kernel(ftp_ref, ftt_ref, final_ref, s_ref):
    offc = ftp_ref[:, 4:5] * _MAXWH
    x1c = ftp_ref[:, 0:1] + offc
    y1c = ftp_ref[:, 1:2] + offc
    x2c = ftp_ref[:, 2:3] + offc
    y2c = ftp_ref[:, 3:4] + offc
    offr = ftt_ref[4:5, :] * _MAXWH
    x1r = ftt_ref[0:1, :] + offr
    y1r = ftt_ref[1:2, :] + offr
    x2r = ftt_ref[2:3, :] + offr
    y2r = ftt_ref[3:4, :] + offr
    areac = (x2c - x1c) * (y2c - y1c)
    arear = (x2r - x1r) * (y2r - y1r)
    xx1 = jnp.maximum(x1c, x1r)
    yy1 = jnp.maximum(y1c, y1r)
    xx2 = jnp.minimum(x2c, x2r)
    yy2 = jnp.minimum(y2c, y2r)
    w = jnp.maximum(xx2 - xx1, 0.0)
    h = jnp.maximum(yy2 - yy1, 0.0)
    inter = w * h
    iou = inter / (areac + arear - inter + 1e-7)
    rowi = lax.broadcasted_iota(jnp.int32, (_CANDP, _CANDP), 0)
    coli = lax.broadcasted_iota(jnp.int32, (_CANDP, _CANDP), 1)
    s_ref[...] = jnp.where((iou > _IOU) & (rowi > coli), 1.0, 0.0)

    def cond(c):
        return c[1]

    def body(c):
        k, _ = c
        sup = lax.dot_general(s_ref[...], k, (((1,), (0,)), ((), ())),
                              preferred_element_type=jnp.float32)
        nk = jnp.where(sup > 0.5, 0.0, 1.0)
        return nk, jnp.any(nk != k)

    k, _ = lax.while_loop(
        cond, body, (jnp.ones((_CANDP, 128), jnp.float32), jnp.bool_(True)))
    kcol = k[:, 0:1]
    tsc = ftp_ref[:, 5:6]
    final_ref[...] = jnp.where((kcol > 0.5) & (tsc > 0.0), tsc, 0.0)


def _detect(x24, x27, x30, x33, interpret=False):
    return pl.pallas_call(
        _detect_kernel,
        out_shape=[
            jax.ShapeDtypeStruct((_NP, 1), jnp.float32),
            jax.ShapeDtypeStruct((_NP, 8), jnp.float32),
        ],
        interpret=interpret,
    )(x24, x27, x30, x33)


def _nms(ftp, ftt, interpret=False):
    return pl.pallas_call(
        _nms_kernel,
        out_shape=jax.ShapeDtypeStruct((_CANDP, 1), jnp.float32),
        scratch_shapes=[pltpu.VMEM((_CANDP, _CANDP), jnp.float32)],
        interpret=interpret,
    )(ftp, ftt)


def _run(x24, x27, x30, x33, interpret=False):
    score, feat = _detect(x24, x27, x30, x33, interpret=interpret)
    ts, ti = lax.top_k(score[:, 0], _CAND)
    ft = feat[ti]
    ftp = jnp.pad(ft, ((0, _CANDP - _CAND), (0, 0)))
    final = _nms(ftp, ftp.T, interpret=interpret)
    det_sc, det_i = lax.top_k(final[:_CAND, 0], _MAXDET)
    d = ft[det_i]
    return jnp.concatenate([d[:, :4], det_sc[:, None], d[:, 4:5]], axis=1)


def kernel(x24, x27, x30, x33):
    return _run(x24, x27, x30, x33)


# speedup vs baseline: 66.2387x; 1.0569x over previous
import jax
import jax.numpy as jnp
from jax import lax
from jax.experimental import pallas as pl
from jax.experimental.pallas import tpu as pltpu

_CONF = 0.3
_IOU = 0.45
_CAND = 1000
_CANDP = 1024
_MAXDET = 300
_MAXWH = 4096.0
_N = 16320
_NP = 16384
_SELP = 304
_PARTS = (12288, 3072, 768, 192)


def _detect_kernel(x24_ref, x27_ref, x30_ref, x33_ref, score_ref, feat_ref):
    off = 0
    for ref, n in zip((x24_ref, x27_ref, x30_ref, x33_ref), _PARTS):
        p = ref[0]
        col = lax.broadcasted_iota(jnp.int32, p.shape, 1)
        obj = p[:, 4:5]
        joint = obj * p
        joint = jnp.where((col >= 5) & (col < 85), joint, -jnp.inf)
        maxv = jnp.max(joint, axis=1, keepdims=True)
        cid = jnp.min(jnp.where(joint == maxv, col - 5, 10**9), axis=1,
                      keepdims=True).astype(jnp.float32)
        score = jnp.where(maxv > _CONF, maxv, 0.0)
        score_ref[off:off + n, :] = score
        x = p[:, 0:1]
        y = p[:, 1:2]
        w = p[:, 2:3]
        h = p[:, 3:4]
        zero = jnp.zeros_like(score)
        feat_ref[off:off + n, :] = jnp.concatenate(
            [x - w / 2.0, y - h / 2.0, x + w / 2.0, y + h / 2.0,
             cid, score, zero, zero], axis=1)
        off += n
    score_ref[_N:, :] = jnp.full((_NP - _N, 1), -1.0, jnp.float32)
    feat_ref[_N:, :] = jnp.zeros((_NP - _N, 8), jnp.float32)


def _nms_kernel(ftp_ref, ftt_ref, out_ref, s_ref, i_ref):
    offc = ftp_ref[:, 4:5] * _MAXWH
    x1c = ftp_ref[:, 0:1] + offc
    y1c = ftp_ref[:, 1:2] + offc
    x2c = ftp_ref[:, 2:3] + offc
    y2c = ftp_ref[:, 3:4] + offc
    offr = ftt_ref[4:5, :] * _MAXWH
    x1r = ftt_ref[0:1, :] + offr
    y1r = ftt_ref[1:2, :] + offr
    x2r = ftt_ref[2:3, :] + offr
    y2r = ftt_ref[3:4, :] + offr
    areac = (x2c - x1c) * (y2c - y1c)
    arear = (x2r - x1r) * (y2r - y1r)
    xx1 = jnp.maximum(x1c, x1r)
    yy1 = jnp.maximum(y1c, y1r)
    xx2 = jnp.minimum(x2c, x2r)
    yy2 = jnp.minimum(y2c, y2r)
    w = jnp.maximum(xx2 - xx1, 0.0)
    h = jnp.maximum(yy2 - yy1, 0.0)
    inter = w * h
    iou = inter / (areac + arear - inter + 1e-7)
    rowi = lax.broadcasted_iota(jnp.int32, (_CANDP, _CANDP), 0)
    coli = lax.broadcasted_iota(jnp.int32, (_CANDP, _CANDP), 1)
    s_ref[...] = jnp.where((iou > _IOU) & (rowi > coli), 1.0, 0.0)

    def cond(c):
        return c[1]

    def body(c):
        k, _ = c
        sup = lax.dot_general(s_ref[...], k, (((1,), (0,)), ((), ())),
                              preferred_element_type=jnp.float32)
        nk = jnp.where(sup > 0.5, 0.0, 1.0)
        return nk, jnp.any(nk != k)

    k, _ = lax.while_loop(
        cond, body, (jnp.ones((_CANDP, 128), jnp.float32), jnp.bool_(True)))
    kcol = k[:, 0:1]
    i_ref[...] = jnp.where(rowi == coli, 1.0, 0.0)
    krow = lax.dot_general(kcol, i_ref[...], (((0,), (0,)), ((), ())),
                           preferred_element_type=jnp.float32)
    tsc = ftp_ref[:, 5:6]
    tsr = ftt_ref[5:6, :]
    fcol = jnp.where((kcol > 0.5) & (tsc > 0.0), tsc, 0.0)
    frow = jnp.where((krow > 0.5) & (tsr > 0.0), tsr, 0.0)
    g = jnp.where((frow > fcol) | ((frow == fcol) & (coli < rowi)), 1.0, 0.0)
    rank = jnp.sum(g, axis=1, keepdims=True)
    p = lax.broadcasted_iota(jnp.int32, (_CANDP, _SELP), 1).astype(jnp.float32)
    rm = jnp.where(rank == p, 1.0, 0.0)
    ft2t = jnp.concatenate([ftt_ref[0:4, :], frow, ftt_ref[4:5, :]], axis=0)
    out_ref[...] = lax.dot_general(ft2t, rm, (((1,), (0,)), ((), ())),
                                   preferred_element_type=jnp.float32,
                                   precision=lax.Precision.HIGHEST)


def _detect(x24, x27, x30, x33, interpret=False):
    return pl.pallas_call(
        _detect_kernel,
        out_shape=[
            jax.ShapeDtypeStruct((_NP, 1), jnp.float32),
            jax.ShapeDtypeStruct((_NP, 8), jnp.float32),
        ],
        interpret=interpret,
    )(x24, x27, x30, x33)


def _nms(ftp, ftt, interpret=False):
    return pl.pallas_call(
        _nms_kernel,
        out_shape=jax.ShapeDtypeStruct((6, _SELP), jnp.float32),
        scratch_shapes=[pltpu.VMEM((_CANDP, _CANDP), jnp.float32),
                        pltpu.VMEM((_CANDP, _CANDP), jnp.float32)],
        interpret=interpret,
    )(ftp, ftt)


def _run(x24, x27, x30, x33, interpret=False):
    score, feat = _detect(x24, x27, x30, x33, interpret=interpret)
    ts, ti = lax.top_k(score[:, 0], _CAND)
    ft = feat[ti]
    ftp = jnp.pad(ft, ((0, _CANDP - _CAND), (0, 0)))
    outt = _nms(ftp, ftp.T, interpret=interpret)
    return outt[:, :_MAXDET].T


def kernel(x24, x27, x30, x33):
    return _run(x24, x27, x30, x33)


# speedup vs baseline: 67.4660x; 1.0185x over previous
import jax
import jax.numpy as jnp
from jax import lax
from jax.experimental import pallas as pl
from jax.experimental.pallas import tpu as pltpu

_CONF = 0.3
_IOU = 0.45
_CAND = 1000
_CANDP = 1024
_MAXDET = 300
_MAXWH = 4096.0
_N = 16320
_NP = 16384
_SELP = 304
_PARTS = (12288, 3072, 768, 192)


def _score_feat(p):
    col = lax.broadcasted_iota(jnp.int32, p.shape, 1)
    obj = p[:, 4:5]
    joint = obj * p
    joint = jnp.where((col >= 5) & (col < 85), joint, -jnp.inf)
    maxv = jnp.max(joint, axis=1, keepdims=True)
    cid = jnp.min(jnp.where(joint == maxv, col - 5, 10**9), axis=1,
                  keepdims=True).astype(jnp.float32)
    score = jnp.where(maxv > _CONF, maxv, 0.0)
    x = p[:, 0:1]
    y = p[:, 1:2]
    w = p[:, 2:3]
    h = p[:, 3:4]
    zero = jnp.zeros_like(score)
    feat = jnp.concatenate(
        [x - w / 2.0, y - h / 2.0, x + w / 2.0, y + h / 2.0,
         cid, score, zero, zero], axis=1)
    return score, feat


def _detect_kernel(x24_ref, x27_ref, x30_ref, x33_ref, score_ref, feat_ref):
    i = pl.program_id(0)

    @pl.when(i < 12)
    def _():
        score, feat = _score_feat(x24_ref[0])
        score_ref[...] = score
        feat_ref[...] = feat

    @pl.when((i >= 12) & (i < 15))
    def _():
        score, feat = _score_feat(x27_ref[0])
        score_ref[...] = score
        feat_ref[...] = feat

    @pl.when(i == 15)
    def _():
        s30, f30 = _score_feat(x30_ref[0])
        s33, f33 = _score_feat(x33_ref[0])
        npad = 1024 - 768 - 192
        score_ref[...] = jnp.concatenate(
            [s30, s33, jnp.full((npad, 1), -1.0, jnp.float32)], axis=0)
        feat_ref[...] = jnp.concatenate(
            [f30, f33, jnp.zeros((npad, 8), jnp.float32)], axis=0)


def _nms_kernel(ftp_ref, ftt_ref, out_ref, s_ref, i_ref):
    offc = ftp_ref[:, 4:5] * _MAXWH
    x1c = ftp_ref[:, 0:1] + offc
    y1c = ftp_ref[:, 1:2] + offc
    x2c = ftp_ref[:, 2:3] + offc
    y2c = ftp_ref[:, 3:4] + offc
    offr = ftt_ref[4:5, :] * _MAXWH
    x1r = ftt_ref[0:1, :] + offr
    y1r = ftt_ref[1:2, :] + offr
    x2r = ftt_ref[2:3, :] + offr
    y2r = ftt_ref[3:4, :] + offr
    areac = (x2c - x1c) * (y2c - y1c)
    arear = (x2r - x1r) * (y2r - y1r)
    xx1 = jnp.maximum(x1c, x1r)
    yy1 = jnp.maximum(y1c, y1r)
    xx2 = jnp.minimum(x2c, x2r)
    yy2 = jnp.minimum(y2c, y2r)
    w = jnp.maximum(xx2 - xx1, 0.0)
    h = jnp.maximum(yy2 - yy1, 0.0)
    inter = w * h
    iou = inter / (areac + arear - inter + 1e-7)
    rowi = lax.broadcasted_iota(jnp.int32, (_CANDP, _CANDP), 0)
    coli = lax.broadcasted_iota(jnp.int32, (_CANDP, _CANDP), 1)
    s_ref[...] = jnp.where((iou > _IOU) & (rowi > coli), 1.0, 0.0)

    def cond(c):
        return c[1]

    def body(c):
        k, _ = c
        sup = lax.dot_general(s_ref[...], k, (((1,), (0,)), ((), ())),
                              preferred_element_type=jnp.float32)
        nk = jnp.where(sup > 0.5, 0.0, 1.0)
        return nk, jnp.any(nk != k)

    k, _ = lax.while_loop(
        cond, body, (jnp.ones((_CANDP, 128), jnp.float32), jnp.bool_(True)))
    kcol = k[:, 0:1]
    i_ref[...] = jnp.where(rowi == coli, 1.0, 0.0)
    krow = lax.dot_general(kcol, i_ref[...], (((0,), (0,)), ((), ())),
                           preferred_element_type=jnp.float32)
    tsc = ftp_ref[:, 5:6]
    tsr = ftt_ref[5:6, :]
    fcol = jnp.where((kcol > 0.5) & (tsc > 0.0), tsc, 0.0)
    frow = jnp.where((krow > 0.5) & (tsr > 0.0), tsr, 0.0)
    g = jnp.where((frow > fcol) | ((frow == fcol) & (coli < rowi)), 1.0, 0.0)
    rank = jnp.sum(g, axis=1, keepdims=True)
    p = lax.broadcasted_iota(jnp.int32, (_CANDP, _SELP), 1).astype(jnp.float32)
    rm = jnp.where(rank == p, 1.0, 0.0)
    ft2t = jnp.concatenate([ftt_ref[0:4, :], frow, ftt_ref[4:5, :]], axis=0)
    out_ref[...] = lax.dot_general(ft2t, rm, (((1,), (0,)), ((), ())),
                                   preferred_element_type=jnp.float32,
                                   precision=lax.Precision.HIGHEST)


def _detect(x24, x27, x30, x33, interpret=False):
    return pl.pallas_call(
        _detect_kernel,
        grid=(16,),
        in_specs=[
            pl.BlockSpec((1, 1024, 85),
                         lambda i: (0, jnp.minimum(i, 11), 0)),
            pl.BlockSpec((1, 1024, 85),
                         lambda i: (0, jnp.clip(i - 12, 0, 2), 0)),
            pl.BlockSpec((1, 768, 85), lambda i: (0, 0, 0)),
            pl.BlockSpec((1, 192, 85), lambda i: (0, 0, 0)),
        ],
        out_specs=[
            pl.BlockSpec((1024, 1), lambda i: (i, 0)),
            pl.BlockSpec((1024, 8), lambda i: (i, 0)),
        ],
        out_shape=[
            jax.ShapeDtypeStruct((_NP, 1), jnp.float32),
            jax.ShapeDtypeStruct((_NP, 8), jnp.float32),
        ],
        interpret=interpret,
    )(x24, x27, x30, x33)


def _nms(ftp, ftt, interpret=False):
    return pl.pallas_call(
        _nms_kernel,
        out_shape=jax.ShapeDtypeStruct((6, _SELP), jnp.float32),
        scratch_shapes=[pltpu.VMEM((_CANDP, _CANDP), jnp.float32),
                        pltpu.VMEM((_CANDP, _CANDP), jnp.float32)],
        interpret=interpret,
    )(ftp, ftt)


def _run(x24, x27, x30, x33, interpret=False):
    score, feat = _detect(x24, x27, x30, x33, interpret=interpret)
    ts, ti = lax.top_k(score[:, 0], _CAND)
    ft = feat[ti]
    ftp = jnp.pad(ft, ((0, _CANDP - _CAND), (0, 0)))
    outt = _nms(ftp, ftp.T, interpret=interpret)
    return outt[:, :_MAXDET].T


def kernel(x24, x27, x30, x33):
    return _run(x24, x27, x30, x33)


# speedup vs baseline: 71.6800x; 1.0625x over previous
import jax
import jax.numpy as jnp
from jax import lax
from jax.experimental import pallas as pl
from jax.experimental.pallas import tpu as pltpu

_CONF = 0.3
_IOU = 0.45
_CAND = 1000
_CANDP = 1024
_MAXDET = 300
_MAXWH = 4096.0
_N = 16320
_NP = 16384
_SELP = 304
_PARTS = (12288, 3072, 768, 192)


def _score_feat(p):
    m = p.shape[0]
    p128 = jnp.concatenate([p, jnp.zeros((m, 128 - 85), jnp.float32)],
                           axis=1)
    pt = jnp.transpose(p128)
    rowi = lax.broadcasted_iota(jnp.int32, (128, m), 0)
    objr = pt[4:5, :]
    jt = jnp.where((rowi >= 5) & (rowi < 85), objr * pt, -jnp.inf)
    maxv = jnp.max(jt, axis=0, keepdims=True)
    cid = jnp.min(jnp.where(jt == maxv, rowi - 5, 10**9), axis=0,
                  keepdims=True).astype(jnp.float32)
    score = jnp.where(maxv > _CONF, maxv, 0.0)
    x = pt[0:1, :]
    y = pt[1:2, :]
    w = pt[2:3, :]
    h = pt[3:4, :]
    zero = jnp.zeros_like(score)
    featt = jnp.concatenate(
        [x - w / 2.0, y - h / 2.0, x + w / 2.0, y + h / 2.0,
         cid, score, zero, zero], axis=0)
    feat = jnp.transpose(featt)
    return feat[:, 5:6], feat


def _detect_kernel(x24_ref, x27_ref, x30_ref, x33_ref, score_ref, feat_ref):
    i = pl.program_id(0)

    @pl.when(i < 12)
    def _():
        score, feat = _score_feat(x24_ref[0])
        score_ref[...] = score
        feat_ref[...] = feat

    @pl.when((i >= 12) & (i < 15))
    def _():
        score, feat = _score_feat(x27_ref[0])
        score_ref[...] = score
        feat_ref[...] = feat

    @pl.when(i == 15)
    def _():
        s30, f30 = _score_feat(x30_ref[0])
        s33, f33 = _score_feat(x33_ref[0])
        npad = 1024 - 768 - 192
        score_ref[...] = jnp.concatenate(
            [s30, s33, jnp.full((npad, 1), -1.0, jnp.float32)], axis=0)
        feat_ref[...] = jnp.concatenate(
            [f30, f33, jnp.zeros((npad, 8), jnp.float32)], axis=0)


def _nms_kernel(ftp_ref, ftt_ref, out_ref, s_ref, i_ref):
    offc = ftp_ref[:, 4:5] * _MAXWH
    x1c = ftp_ref[:, 0:1] + offc
    y1c = ftp_ref[:, 1:2] + offc
    x2c = ftp_ref[:, 2:3] + offc
    y2c = ftp_ref[:, 3:4] + offc
    offr = ftt_ref[4:5, :] * _MAXWH
    x1r = ftt_ref[0:1, :] + offr
    y1r = ftt_ref[1:2, :] + offr
    x2r = ftt_ref[2:3, :] + offr
    y2r = ftt_ref[3:4, :] + offr
    areac = (x2c - x1c) * (y2c - y1c)
    arear = (x2r - x1r) * (y2r - y1r)
    xx1 = jnp.maximum(x1c, x1r)
    yy1 = jnp.maximum(y1c, y1r)
    xx2 = jnp.minimum(x2c, x2r)
    yy2 = jnp.minimum(y2c, y2r)
    w = jnp.maximum(xx2 - xx1, 0.0)
    h = jnp.maximum(yy2 - yy1, 0.0)
    inter = w * h
    iou = inter / (areac + arear - inter + 1e-7)
    rowi = lax.broadcasted_iota(jnp.int32, (_CANDP, _CANDP), 0)
    coli = lax.broadcasted_iota(jnp.int32, (_CANDP, _CANDP), 1)
    s_ref[...] = jnp.where((iou > _IOU) & (rowi > coli), 1.0, 0.0)

    def cond(c):
        return c[1]

    def body(c):
        k, _ = c
        sup = lax.dot_general(s_ref[...], k, (((1,), (0,)), ((), ())),
                              preferred_element_type=jnp.float32)
        nk = jnp.where(sup > 0.5, 0.0, 1.0)
        return nk, jnp.any(nk != k)

    k, _ = lax.while_loop(
        cond, body, (jnp.ones((_CANDP, 128), jnp.float32), jnp.bool_(True)))
    kcol = k[:, 0:1]
    i_ref[...] = jnp.where(rowi == coli, 1.0, 0.0)
    krow = lax.dot_general(kcol, i_ref[...], (((0,), (0,)), ((), ())),
                           preferred_element_type=jnp.float32)
    tsc = ftp_ref[:, 5:6]
    tsr = ftt_ref[5:6, :]
    fcol = jnp.where((kcol > 0.5) & (tsc > 0.0), tsc, 0.0)
    frow = jnp.where((krow > 0.5) & (tsr > 0.0), tsr, 0.0)
    g = jnp.where((frow > fcol) | ((frow == fcol) & (coli < rowi)), 1.0, 0.0)
    rank = jnp.sum(g, axis=1, keepdims=True)
    p = lax.broadcasted_iota(jnp.int32, (_CANDP, _SELP), 1).astype(jnp.float32)
    rm = jnp.where(rank == p, 1.0, 0.0)
    ft2t = jnp.concatenate([ftt_ref[0:4, :], frow, ftt_ref[4:5, :]], axis=0)
    out_ref[...] = lax.dot_general(ft2t, rm, (((1,), (0,)), ((), ())),
                                   preferred_element_type=jnp.float32,
                                   precision=lax.Precision.HIGHEST)


def _detect(x24, x27, x30, x33, interpret=False):
    return pl.pallas_call(
        _detect_kernel,
        grid=(16,),
        in_specs=[
            pl.BlockSpec((1, 1024, 85),
                         lambda i: (0, jnp.minimum(i, 11), 0)),
            pl.BlockSpec((1, 1024, 85),
                         lambda i: (0, jnp.clip(i - 12, 0, 2), 0)),
            pl.BlockSpec((1, 768, 85), lambda i: (0, 0, 0)),
            pl.BlockSpec((1, 192, 85), lambda i: (0, 0, 0)),
        ],
        out_specs=[
            pl.BlockSpec((1024, 1), lambda i: (i, 0)),
            pl.BlockSpec((1024, 8), lambda i: (i, 0)),
        ],
        out_shape=[
            jax.ShapeDtypeStruct((_NP, 1), jnp.float32),
            jax.ShapeDtypeStruct((_NP, 8), jnp.float32),
        ],
        interpret=interpret,
    )(x24, x27, x30, x33)


def _nms(ftp, ftt, interpret=False):
    return pl.pallas_call(
        _nms_kernel,
        out_shape=jax.ShapeDtypeStruct((6, _SELP), jnp.float32),
        scratch_shapes=[pltpu.VMEM((_CANDP, _CANDP), jnp.float32),
                        pltpu.VMEM((_CANDP, _CANDP), jnp.float32)],
        interpret=interpret,
    )(ftp, ftt)


def _run(x24, x27, x30, x33, interpret=False):
    score, feat = _detect(x24, x27, x30, x33, interpret=interpret)
    ts, ti = lax.top_k(score[:, 0], _CAND)
    ft = feat[ti]
    ftp = jnp.pad(ft, ((0, _CANDP - _CAND), (0, 0)))
    outt = _nms(ftp, ftp.T, interpret=interpret)
    return outt[:, :_MAXDET].T


def kernel(x24, x27, x30, x33):
    return _run(x24, x27, x30, x33)
